# baseline (speedup 1.0000x reference)
import functools

import jax
import jax.numpy as jnp
from jax import lax
from jax.experimental import pallas as pl
from jax.experimental.pallas import tpu as pltpu
from jax.experimental.pallas import tpu_sc as plsc

N_CORES = 2
N_TILES = 16
LANES = 16
CHUNK = 128
HALF = 64


def _mm_body(x_ref, w_ref, o_ref):
    o_ref[...] = jnp.dot(x_ref[...], w_ref[0],
                         preferred_element_type=jnp.float32)


def _support_colsplit(x, w):
    n = x.shape[0]
    rb = 1000
    nrb = n // rb
    ws = w.reshape(w.shape[0], N_CORES, HALF).transpose(1, 0, 2)
    return pl.pallas_call(
        _mm_body,
        grid=(N_CORES, nrb),
        in_specs=[
            pl.BlockSpec((rb, x.shape[1]), lambda h, i: (i, 0)),
            pl.BlockSpec((1, x.shape[1], HALF), lambda h, i: (h, 0, 0)),
        ],
        out_specs=pl.BlockSpec((rb, HALF), lambda h, i: (h * nrb + i, 0)),
        out_shape=jax.ShapeDtypeStruct((N_CORES * n, HALF), jnp.float32),
    )(x, ws)


def _make_agg(n_nodes, nchunk):
    rows_per_tile = n_nodes // N_TILES
    epil = CHUNK - 3
    n_init = rows_per_tile // epil
    mesh = plsc.VectorSubcoreMesh(core_axis_name="c", subcore_axis_name="s")

    @functools.partial(
        pl.kernel,
        out_type=jax.ShapeDtypeStruct((n_nodes, 2 * HALF), jnp.float32),
        mesh=mesh,
        compiler_params=pltpu.CompilerParams(
            use_tc_tiling_on_sc=False, needs_layout_passes=False),
        scratch_types=[
            pltpu.VMEM((nchunk, CHUNK), jnp.int32),
            pltpu.VMEM((nchunk, CHUNK), jnp.float32),
            pltpu.VMEM((2, CHUNK), jnp.int32),
            pltpu.VMEM((2, CHUNK), jnp.int32),
            pltpu.VMEM((2, CHUNK, HALF), jnp.float32),
            pltpu.VMEM((2, CHUNK, HALF), jnp.float32),
            pltpu.VMEM((2 * HALF,), jnp.float32),
            pltpu.VMEM_SHARED((n_nodes, HALF), jnp.float32),
            pltpu.SemaphoreType.DMA,
            pltpu.SemaphoreType.DMA,
            pltpu.SemaphoreType.DMA,
            pltpu.SemaphoreType.DMA,
        ],
    )
    def agg(support_ref, packed_ref, val_ref, bias_ref, out_ref,
            packed_v, val_v, sbuf, dbuf, rin, rout, bias_v, acc,
            gs0, gs1, ss0, ss1):
        c = lax.axis_index("c")
        sid = lax.axis_index("s")

        pltpu.sync_copy(packed_ref.at[sid], packed_v)
        pltpu.sync_copy(val_ref.at[sid], val_v)
        pltpu.sync_copy(bias_ref, bias_v)

        coff = c * n_nodes

        def unpack_src(j, b):
            for k in range(CHUNK // LANES):
                sl = pl.ds(k * LANES, LANES)
                sbuf[b, sl] = (packed_v[j, sl] & 0xFFFF) + coff

        def unpack_dst(j, b):
            for k in range(CHUNK // LANES):
                sl = pl.ds(k * LANES, LANES)
                dbuf[b, sl] = lax.shift_right_logical(packed_v[j, sl], 16)

        bvs = [bias_v[pl.ds(c * HALF + k * LANES, LANES)]
               for k in range(HALF // LANES)]
        def bias_row(r, carry):
            for k in range(HALF // LANES):
                rin[0, r, pl.ds(k * LANES, LANES)] = bvs[k]
            return carry
        lax.fori_loop(0, epil, bias_row, None)
        base = sid * rows_per_tile
        for k in range(n_init):
            pltpu.sync_copy(rin.at[0, pl.ds(0, epil)],
                            acc.at[pl.ds(base + k * epil, epil)])
        plsc.subcore_barrier()

        gsems = (gs0, gs1)
        ssems = (ss0, ss1)
        for b in range(2):
            unpack_src(b, b)
            pltpu.async_copy(support_ref.at[sbuf.at[b]], rin.at[b], gsems[b])

        def pipe_body(j2, carry):
            for b in range(2):
                j = 2 * j2 + b
                pltpu.make_async_copy(
                    support_ref.at[sbuf.at[b]], rin.at[b], gsems[b]).wait()

                @pl.when(j2 >= 1)
                def _():
                    pltpu.make_async_copy(
                        rout.at[b], acc.at[dbuf.at[b]], ssems[b]).wait()

                unpack_dst(j, b)

                def scale_edge(e, carry2):
                    vb = plsc.load_gather(
                        val_v, [jnp.full((LANES,), j, jnp.int32),
                                jnp.full((LANES,), e, jnp.int32)])
                    for k in range(HALF // LANES):
                        sl = pl.ds(k * LANES, LANES)
                        rout[b, e, sl] = rin[b, e, sl] * vb
                    return carry2
                lax.fori_loop(0, CHUNK, scale_edge, None)

                pltpu.async_copy(rout.at[b], acc.at[dbuf.at[b]], ssems[b],
                                 add=True)
                unpack_src(jnp.minimum(j + 2, nchunk - 1), b)
                pltpu.async_copy(
                    support_ref.at[sbuf.at[b]], rin.at[b], gsems[b])
            return carry
        lax.fori_loop(0, nchunk // 2, pipe_body, None)
        for b in range(2):
            pltpu.make_async_copy(
                support_ref.at[sbuf.at[b]], rin.at[b], gsems[b]).wait()
            pltpu.make_async_copy(
                rout.at[b], acc.at[dbuf.at[b]], ssems[b]).wait()
        plsc.subcore_barrier()

        pltpu.sync_copy(
            acc.at[pl.ds(base, rows_per_tile)],
            out_ref.at[pl.ds(base, rows_per_tile), pl.ds(c * HALF, HALF)])

    return agg


def kernel(edge_index, adj_values, input_feature, weight, bias):
    n_nodes = input_feature.shape[0]
    n_edges = adj_values.shape[0]
    src = edge_index[0].astype(jnp.int32)
    dst = edge_index[1].astype(jnp.int32)

    nch = -(-n_edges // (N_TILES * CHUNK))
    nch += nch % 2
    per_tile = nch * CHUNK
    e_pad = N_TILES * per_tile
    pad = e_pad - n_edges
    nchunk = per_tile // CHUNK
    packed = jnp.pad((dst << 16) | src, (0, pad)).reshape(
        N_TILES, nchunk, CHUNK)
    val_p = jnp.pad(adj_values, (0, pad)).reshape(N_TILES, nchunk, CHUNK)

    support = _support_colsplit(input_feature, weight)
    agg = _make_agg(n_nodes, nchunk)
    return agg(support, packed, val_p, bias)


# speedup vs baseline: 1.8799x; 1.8799x over previous
import functools

import jax
import jax.numpy as jnp
from jax import lax
from jax.experimental import pallas as pl
from jax.experimental.pallas import tpu as pltpu
from jax.experimental.pallas import tpu_sc as plsc

N_CORES = 2
N_TILES = 16
LANES = 16
CHUNK = 128
HALF = 64


def _mm_body(x_ref, w_ref, o_ref):
    o_ref[...] = jnp.dot(x_ref[...], w_ref[0],
                         preferred_element_type=jnp.float32)


def _support_colsplit(x, w):
    n = x.shape[0]
    rb = 1000
    nrb = n // rb
    ws = w.reshape(w.shape[0], N_CORES, HALF).transpose(1, 0, 2)
    return pl.pallas_call(
        _mm_body,
        grid=(N_CORES, nrb),
        in_specs=[
            pl.BlockSpec((rb, x.shape[1]), lambda h, i: (i, 0)),
            pl.BlockSpec((1, x.shape[1], HALF), lambda h, i: (h, 0, 0)),
        ],
        out_specs=pl.BlockSpec((rb, HALF), lambda h, i: (h * nrb + i, 0)),
        out_shape=jax.ShapeDtypeStruct((N_CORES * n, HALF), jnp.float32),
    )(x, ws)


def _make_agg(n_nodes, nchunk):
    rows_per_tile = n_nodes // N_TILES
    epil = CHUNK - 3
    n_init = rows_per_tile // epil
    mesh = plsc.VectorSubcoreMesh(core_axis_name="c", subcore_axis_name="s")

    @functools.partial(
        pl.kernel,
        out_type=jax.ShapeDtypeStruct((n_nodes, 2 * HALF), jnp.float32),
        mesh=mesh,
        compiler_params=pltpu.CompilerParams(
            use_tc_tiling_on_sc=False, needs_layout_passes=False),
        scratch_types=[
            pltpu.VMEM((nchunk, CHUNK), jnp.int32),
            pltpu.VMEM((nchunk, CHUNK), jnp.float32),
            pltpu.VMEM((2, CHUNK), jnp.int32),
            pltpu.VMEM((2, CHUNK), jnp.int32),
            pltpu.VMEM((2, CHUNK, HALF), jnp.float32),
            pltpu.VMEM((2, CHUNK, HALF), jnp.float32),
            pltpu.VMEM((2 * HALF,), jnp.float32),
            pltpu.VMEM_SHARED((n_nodes, HALF), jnp.float32),
            pltpu.SemaphoreType.DMA,
            pltpu.SemaphoreType.DMA,
            pltpu.SemaphoreType.DMA,
            pltpu.SemaphoreType.DMA,
        ],
    )
    def agg(support_ref, packed_ref, val_ref, bias_ref, out_ref,
            packed_v, val_v, sbuf, dbuf, rin, rout, bias_v, acc,
            gs0, gs1, ss0, ss1):
        c = lax.axis_index("c")
        sid = lax.axis_index("s")

        pltpu.sync_copy(packed_ref.at[sid], packed_v)
        pltpu.sync_copy(val_ref.at[sid], val_v)
        pltpu.sync_copy(bias_ref, bias_v)

        coff = c * n_nodes

        def unpack_src(j, b):
            for k in range(CHUNK // LANES):
                sl = pl.ds(k * LANES, LANES)
                sbuf[b, sl] = (packed_v[j, sl] & 0xFFFF) + coff

        def unpack_dst(j, b):
            for k in range(CHUNK // LANES):
                sl = pl.ds(k * LANES, LANES)
                dbuf[b, sl] = lax.shift_right_logical(packed_v[j, sl], 16)

        bvs = [bias_v[pl.ds(c * HALF + k * LANES, LANES)]
               for k in range(HALF // LANES)]
        def bias_row(r, carry):
            for k in range(HALF // LANES):
                rin[0, r, pl.ds(k * LANES, LANES)] = bvs[k]
            return carry
        lax.fori_loop(0, epil, bias_row, None)
        base = sid * rows_per_tile
        for k in range(n_init):
            pltpu.sync_copy(rin.at[0, pl.ds(0, epil)],
                            acc.at[pl.ds(base + k * epil, epil)])
        plsc.subcore_barrier()

        gsems = (gs0, gs1)
        ssems = (ss0, ss1)
        for b in range(2):
            unpack_src(b, b)
            pltpu.async_copy(support_ref.at[sbuf.at[b]], rin.at[b], gsems[b])

        def pipe_body(j2, carry):
            for b in range(2):
                j = 2 * j2 + b
                pltpu.make_async_copy(
                    support_ref.at[sbuf.at[b]], rin.at[b], gsems[b]).wait()

                @pl.when(j2 >= 1)
                def _():
                    pltpu.make_async_copy(
                        rout.at[b], acc.at[dbuf.at[b]], ssems[b]).wait()

                unpack_dst(j, b)

                def scale_grp(g, carry2):
                    vvec = val_v[j, pl.ds(g * LANES, LANES)]
                    e0 = g * LANES
                    for i in range(LANES):
                        vb = vvec.at[jnp.full((LANES,), i, jnp.int32)].get(
                            mode="promise_in_bounds")
                        for k in range(HALF // LANES):
                            sl = pl.ds(k * LANES, LANES)
                            rout[b, e0 + i, sl] = rin[b, e0 + i, sl] * vb
                    return carry2
                lax.fori_loop(0, CHUNK // LANES, scale_grp, None)

                pltpu.async_copy(rout.at[b], acc.at[dbuf.at[b]], ssems[b],
                                 add=True)
                unpack_src(jnp.minimum(j + 2, nchunk - 1), b)
                pltpu.async_copy(
                    support_ref.at[sbuf.at[b]], rin.at[b], gsems[b])
            return carry
        lax.fori_loop(0, nchunk // 2, pipe_body, None)
        for b in range(2):
            pltpu.make_async_copy(
                support_ref.at[sbuf.at[b]], rin.at[b], gsems[b]).wait()
            pltpu.make_async_copy(
                rout.at[b], acc.at[dbuf.at[b]], ssems[b]).wait()
        plsc.subcore_barrier()

        pltpu.sync_copy(
            acc.at[pl.ds(base, rows_per_tile)],
            out_ref.at[pl.ds(base, rows_per_tile), pl.ds(c * HALF, HALF)])

    return agg


def kernel(edge_index, adj_values, input_feature, weight, bias):
    n_nodes = input_feature.shape[0]
    n_edges = adj_values.shape[0]
    src = edge_index[0].astype(jnp.int32)
    dst = edge_index[1].astype(jnp.int32)

    nch = -(-n_edges // (N_TILES * CHUNK))
    nch += nch % 2
    per_tile = nch * CHUNK
    e_pad = N_TILES * per_tile
    pad = e_pad - n_edges
    nchunk = per_tile // CHUNK
    packed = jnp.pad((dst << 16) | src, (0, pad)).reshape(
        N_TILES, nchunk, CHUNK)
    val_p = jnp.pad(adj_values, (0, pad)).reshape(N_TILES, nchunk, CHUNK)

    support = _support_colsplit(input_feature, weight)
    agg = _make_agg(n_nodes, nchunk)
    return agg(support, packed, val_p, bias)


# speedup vs baseline: 1.8899x; 1.0053x over previous
import functools

import jax
import jax.numpy as jnp
from jax import lax
from jax.experimental import pallas as pl
from jax.experimental.pallas import tpu as pltpu
from jax.experimental.pallas import tpu_sc as plsc

N_CORES = 2
N_TILES = 16
LANES = 16
CHUNK = 128
HALF = 64


def _mm_body(x_ref, w_ref, o_ref):
    o_ref[...] = jnp.dot(x_ref[...], w_ref[0],
                         preferred_element_type=jnp.float32)


def _support_colsplit(x, w):
    n = x.shape[0]
    rb = 1000
    nrb = n // rb
    ws = w.reshape(w.shape[0], N_CORES, HALF).transpose(1, 0, 2)
    return pl.pallas_call(
        _mm_body,
        grid=(N_CORES, nrb),
        in_specs=[
            pl.BlockSpec((rb, x.shape[1]), lambda h, i: (i, 0)),
            pl.BlockSpec((1, x.shape[1], HALF), lambda h, i: (h, 0, 0)),
        ],
        out_specs=pl.BlockSpec((rb, HALF), lambda h, i: (h * nrb + i, 0)),
        out_shape=jax.ShapeDtypeStruct((N_CORES * n, HALF), jnp.float32),
    )(x, ws)


def _make_agg(n_nodes, nchunk):
    rows_per_tile = n_nodes // N_TILES
    epil = CHUNK - 3
    n_init = rows_per_tile // epil
    mesh = plsc.VectorSubcoreMesh(core_axis_name="c", subcore_axis_name="s")

    @functools.partial(
        pl.kernel,
        out_type=jax.ShapeDtypeStruct((n_nodes, 2 * HALF), jnp.float32),
        mesh=mesh,
        compiler_params=pltpu.CompilerParams(
            use_tc_tiling_on_sc=False, needs_layout_passes=False),
        scratch_types=[
            pltpu.VMEM((nchunk, CHUNK), jnp.int32),
            pltpu.VMEM((nchunk, CHUNK), jnp.float32),
            pltpu.VMEM((2, CHUNK), jnp.int32),
            pltpu.VMEM((2, CHUNK), jnp.int32),
            pltpu.VMEM((2, CHUNK, HALF), jnp.float32),
            pltpu.VMEM((2, CHUNK, HALF), jnp.float32),
            pltpu.VMEM((2 * HALF,), jnp.float32),
            pltpu.VMEM_SHARED((n_nodes, HALF), jnp.float32),
            pltpu.SemaphoreType.DMA,
            pltpu.SemaphoreType.DMA,
            pltpu.SemaphoreType.DMA,
            pltpu.SemaphoreType.DMA,
        ],
    )
    def agg(support_ref, packed_ref, val_ref, bias_ref, out_ref,
            packed_v, val_v, sbuf, dbuf, rin, rout, bias_v, acc,
            gs0, gs1, ss0, ss1):
        c = lax.axis_index("c")
        sid = lax.axis_index("s")

        pltpu.sync_copy(packed_ref.at[sid], packed_v)
        pltpu.sync_copy(val_ref.at[sid], val_v)
        pltpu.sync_copy(bias_ref, bias_v)

        coff = c * n_nodes

        def unpack_src(j, b):
            for k in range(CHUNK // LANES):
                sl = pl.ds(k * LANES, LANES)
                sbuf[b, sl] = (packed_v[j, sl] & 0xFFFF) + coff

        def unpack_dst(j, b):
            for k in range(CHUNK // LANES):
                sl = pl.ds(k * LANES, LANES)
                dbuf[b, sl] = lax.shift_right_logical(packed_v[j, sl], 16)

        bvs = [bias_v[pl.ds(c * HALF + k * LANES, LANES)]
               for k in range(HALF // LANES)]
        def bias_row(r, carry):
            for k in range(HALF // LANES):
                rin[0, r, pl.ds(k * LANES, LANES)] = bvs[k]
            return carry
        lax.fori_loop(0, epil, bias_row, None)
        base = sid * rows_per_tile
        for k in range(n_init):
            pltpu.sync_copy(rin.at[0, pl.ds(0, epil)],
                            acc.at[pl.ds(base + k * epil, epil)])
        plsc.subcore_barrier()

        gsems = (gs0, gs1)
        ssems = (ss0, ss1)
        for b in range(2):
            unpack_src(b, b)
            pltpu.async_copy(support_ref.at[sbuf.at[b]], rin.at[b], gsems[b])

        def pipe_body(j2, carry):
            for b in range(2):
                j = 2 * j2 + b

                @pl.when(j2 >= 1)
                def _():
                    pltpu.make_async_copy(
                        rout.at[b], acc.at[dbuf.at[b]], ssems[b]).wait()

                unpack_dst(j, b)
                pltpu.make_async_copy(
                    support_ref.at[sbuf.at[b]], rin.at[b], gsems[b]).wait()

                for g in range(CHUNK // LANES):
                    e0 = g * LANES
                    vvec = val_v[j, pl.ds(e0, LANES)]
                    for i in range(LANES):
                        vb = jnp.broadcast_to(vvec[i], (LANES,))
                        for k in range(HALF // LANES):
                            sl = pl.ds(k * LANES, LANES)
                            rout[b, e0 + i, sl] = rin[b, e0 + i, sl] * vb

                pltpu.async_copy(rout.at[b], acc.at[dbuf.at[b]], ssems[b],
                                 add=True)
                unpack_src(jnp.minimum(j + 2, nchunk - 1), b)
                pltpu.async_copy(
                    support_ref.at[sbuf.at[b]], rin.at[b], gsems[b])
            return carry
        lax.fori_loop(0, nchunk // 2, pipe_body, None)
        for b in range(2):
            pltpu.make_async_copy(
                support_ref.at[sbuf.at[b]], rin.at[b], gsems[b]).wait()
            pltpu.make_async_copy(
                rout.at[b], acc.at[dbuf.at[b]], ssems[b]).wait()
        plsc.subcore_barrier()

        pltpu.sync_copy(
            acc.at[pl.ds(base, rows_per_tile)],
            out_ref.at[pl.ds(base, rows_per_tile), pl.ds(c * HALF, HALF)])

    return agg


def kernel(edge_index, adj_values, input_feature, weight, bias):
    n_nodes = input_feature.shape[0]
    n_edges = adj_values.shape[0]
    src = edge_index[0].astype(jnp.int32)
    dst = edge_index[1].astype(jnp.int32)

    nch = -(-n_edges // (N_TILES * CHUNK))
    nch += nch % 2
    per_tile = nch * CHUNK
    e_pad = N_TILES * per_tile
    pad = e_pad - n_edges
    nchunk = per_tile // CHUNK
    packed = jnp.pad((dst << 16) | src, (0, pad)).reshape(
        N_TILES, nchunk, CHUNK)
    val_p = jnp.pad(adj_values, (0, pad)).reshape(N_TILES, nchunk, CHUNK)

    support = _support_colsplit(input_feature, weight)
    agg = _make_agg(n_nodes, nchunk)
    return agg(support, packed, val_p, bias)
